# baseline (speedup 1.0000x reference)
import jax
import jax.numpy as jnp
from jax.experimental import pallas as pl
from jax.experimental.pallas import tpu as pltpu

C = 8
N_NODES = 32
D_HID = 128
E_EDGES = 128
D_IN = 1024
B = 32
T = 30
N_F = B * T


def _branch_kernel(x_ref, emb_ref, es_row_ref, es_col_ref, ed_row_ref,
                   ed_col_ref, win_ref, wmsg_ref, wself_ref, watt_ref,
                   wout_ref, logits_ref, v_ref, c1_ref, c2_ref, bp_ref):
    f32 = jnp.float32
    bf16 = jnp.bfloat16
    iota_ne = jax.lax.broadcasted_iota(jnp.int32, (N_NODES, E_EDGES), 0)
    iota_en = jax.lax.broadcasted_iota(jnp.int32, (E_EDGES, N_NODES), 1)

    logits_acc = jnp.zeros((N_F, 1), f32)
    v_acc = jnp.zeros((N_F, C), f32)

    for c in range(C):
        src_row = es_row_ref[c]
        src_col = es_col_ref[c]
        dst_row = ed_row_ref[c]
        dst_col = ed_col_ref[c]

        Dh = (iota_ne == dst_row).astype(bf16)
        Sh = (iota_ne == src_row).astype(bf16)
        ShT = (iota_en == src_col).astype(bf16)
        A = jnp.dot(Dh, ShT, preferred_element_type=f32)
        deg = jnp.sum(A, axis=1, keepdims=True)
        d31 = (dst_col == (N_NODES - 1)).astype(bf16)
        a31 = jnp.dot(Sh, d31, preferred_element_type=f32)

        emb = emb_ref[c]
        wm1 = wmsg_ref[c, 0]
        wm2 = wmsg_ref[c, 1]
        ws1 = wself_ref[c, 0]
        ws2 = wself_ref[c, 1]
        Aemb = jnp.dot(A.astype(bf16), emb.astype(bf16),
                       preferred_element_type=f32)
        bnode = (jnp.dot(Aemb.astype(bf16), wm1, preferred_element_type=f32)
                 + jnp.dot(emb.astype(bf16), ws1,
                           preferred_element_type=f32))

        proj = jnp.tanh(jnp.dot(x_ref[:], win_ref[c],
                                preferred_element_type=f32))
        proj_b = proj.astype(bf16)
        P1 = jnp.dot(proj_b, wm1, preferred_element_type=f32)
        P2 = jnp.dot(proj_b, ws1, preferred_element_type=f32)

        c1 = a31 * deg
        bp = a31 * bnode
        p = jnp.int32(0)
        for n in range(N_NODES):
            an = a31[n, 0]

            @pl.when(an > 0)
            def _(n=n, p=p):
                c1_ref[pl.ds(p, 1), :] = c1[n:n + 1, :]
                c2_ref[pl.ds(p, 1), :] = a31[n:n + 1, :]
                bp_ref[pl.ds(p, 1), :] = bp[n:n + 1, :]

            p = p + (an > 0).astype(jnp.int32)

        def body(i, g):
            c1i = c1_ref[pl.ds(i, 1), :]
            c2i = c2_ref[pl.ds(i, 1), :]
            bi = bp_ref[pl.ds(i, 1), :]
            return g + jax.nn.relu(c1i * P1 + c2i * P2 + bi)

        g = jax.lax.fori_loop(0, p, body, jnp.zeros((N_F, D_HID), f32))
        s = jax.nn.relu(deg[N_NODES - 1, 0] * P1 + P2
                        + bnode[N_NODES - 1:N_NODES, :])
        enc = jax.nn.relu(jnp.dot(g.astype(bf16), wm2,
                                  preferred_element_type=f32)
                          + jnp.dot(s.astype(bf16), ws2,
                                    preferred_element_type=f32))
        enc_b = enc.astype(bf16)

        logits_acc = logits_acc + jnp.dot(enc_b, watt_ref[c],
                                          preferred_element_type=f32)
        v_acc = v_acc + jnp.dot(enc_b, wout_ref[c],
                                preferred_element_type=f32)

    logits_ref[:] = logits_acc
    v_ref[:] = v_acc


def _head_kernel(lg_ref, mask_ref, v_ref, bout_ref, out_ref):
    lg = jnp.where(mask_ref[:] > 0, lg_ref[:], jnp.float32(-1e9))
    m = jnp.max(lg, axis=1, keepdims=True)
    e = jnp.exp(lg - m)
    attn = e / jnp.sum(e, axis=1, keepdims=True)
    cols = []
    for j in range(C):
        vj = v_ref[:, :, j]
        cols.append(jnp.sum(attn * vj, axis=1, keepdims=True))
    out_ref[:] = jnp.concatenate(cols, axis=1) + bout_ref[:]


def kernel(sensor_seq, mask, node_emb, edge_src, edge_dst, W_in, W_msg,
           W_self, w_att, W_out, b_out):
    bf16 = jnp.bfloat16
    x = sensor_seq.reshape(N_F, D_IN).astype(bf16)
    es = edge_src.astype(jnp.int32)
    ed = edge_dst.astype(jnp.int32)
    es_row = es.reshape(C, 1, E_EDGES)
    es_col = es.reshape(C, E_EDGES, 1)
    ed_row = ed.reshape(C, 1, E_EDGES)
    ed_col = ed.reshape(C, E_EDGES, 1)
    watt = w_att.reshape(C, D_HID, 1).astype(bf16)
    wout = W_out.reshape(C, D_HID, C).astype(bf16)

    logits, v = pl.pallas_call(
        _branch_kernel,
        out_shape=[
            jax.ShapeDtypeStruct((N_F, 1), jnp.float32),
            jax.ShapeDtypeStruct((N_F, C), jnp.float32),
        ],
        scratch_shapes=[
            pltpu.VMEM((N_NODES, 1), jnp.float32),
            pltpu.VMEM((N_NODES, 1), jnp.float32),
            pltpu.VMEM((N_NODES, D_HID), jnp.float32),
        ],
    )(x, node_emb, es_row, es_col, ed_row, ed_col, W_in.astype(bf16),
      W_msg.astype(bf16), W_self.astype(bf16), watt, wout)

    lg3 = logits.reshape(B, T)
    v3 = v.reshape(B, T, C)
    out = pl.pallas_call(
        _head_kernel,
        out_shape=jax.ShapeDtypeStruct((B, C), jnp.float32),
    )(lg3, mask, v3, b_out.reshape(1, C))
    return out


# speedup vs baseline: 1.2342x; 1.2342x over previous
import jax
import jax.numpy as jnp
from jax.experimental import pallas as pl
from jax.experimental.pallas import tpu as pltpu

C = 8
N_NODES = 32
D_HID = 128
E_EDGES = 128
D_IN = 1024
B = 32
T = 30
N_F = B * T


def _branch_kernel(x_ref, emb_ref, es_row_ref, es_col_ref, ed_row_ref,
                   ed_col_ref, win_ref, wmsg_ref, wself_ref, watt_ref,
                   wout_ref, mask_ref, bout_ref, out_ref, c1_ref, c2_ref,
                   bp_ref):
    f32 = jnp.float32
    iota_ne = jax.lax.broadcasted_iota(jnp.int32, (N_NODES, E_EDGES), 0)
    iota_en = jax.lax.broadcasted_iota(jnp.int32, (E_EDGES, N_NODES), 1)

    logits_acc = jnp.zeros((1, N_F), f32)
    v_acc = jnp.zeros((N_F, C), f32)

    for c in range(C):
        src_row = es_row_ref[c]
        src_col = es_col_ref[c]
        dst_row = ed_row_ref[c]
        dst_col = ed_col_ref[c]

        Dh = (iota_ne == dst_row).astype(f32)
        Sh = (iota_ne == src_row).astype(f32)
        ShT = (iota_en == src_col).astype(f32)
        A = jnp.dot(Dh, ShT, preferred_element_type=f32)
        deg = jnp.sum(A, axis=1, keepdims=True)
        d31 = (dst_col == (N_NODES - 1)).astype(f32)
        a31 = jnp.dot(Sh, d31, preferred_element_type=f32)

        emb = emb_ref[c]
        wm1 = wmsg_ref[c, 0]
        wm2 = wmsg_ref[c, 1]
        ws1 = wself_ref[c, 0]
        ws2 = wself_ref[c, 1]
        Aemb = jnp.dot(A, emb, preferred_element_type=f32)
        bnode = (jnp.dot(Aemb, wm1, preferred_element_type=f32)
                 + jnp.dot(emb, ws1, preferred_element_type=f32))

        proj = jnp.tanh(jnp.dot(x_ref[:], win_ref[c],
                                preferred_element_type=f32))
        P1 = jnp.dot(proj, wm1, preferred_element_type=f32)
        P2 = jnp.dot(proj, ws1, preferred_element_type=f32)

        c1 = a31 * deg
        bp = a31 * bnode
        p = jnp.int32(0)
        for n in range(N_NODES):
            an = a31[n, 0]

            @pl.when(an > 0)
            def _(n=n, p=p):
                c1_ref[pl.ds(p, 1), :] = c1[n:n + 1, :]
                c2_ref[pl.ds(p, 1), :] = a31[n:n + 1, :]
                bp_ref[pl.ds(p, 1), :] = bp[n:n + 1, :]

            p = p + (an > 0).astype(jnp.int32)

        def body(i, g):
            c1i = c1_ref[pl.ds(i, 1), :]
            c2i = c2_ref[pl.ds(i, 1), :]
            bi = bp_ref[pl.ds(i, 1), :]
            return g + jax.nn.relu(c1i * P1 + c2i * P2 + bi)

        g = jax.lax.fori_loop(0, p, body, jnp.zeros((N_F, D_HID), f32))
        s = jax.nn.relu(deg[N_NODES - 1, 0] * P1 + P2
                        + bnode[N_NODES - 1:N_NODES, :])
        enc = jax.nn.relu(jnp.dot(g, wm2, preferred_element_type=f32)
                          + jnp.dot(s, ws2, preferred_element_type=f32))

        logits_acc = logits_acc + jax.lax.dot_general(
            watt_ref[c], enc, (((1,), (1,)), ((), ())),
            preferred_element_type=f32)
        v_acc = v_acc + jnp.dot(enc, wout_ref[c],
                                preferred_element_type=f32)

    row_i = jax.lax.broadcasted_iota(jnp.int32, (B, N_F), 0)
    col_i = jax.lax.broadcasted_iota(jnp.int32, (B, N_F), 1)
    seg = (col_i // T) == row_i
    valid = seg & (mask_ref[:] > 0)
    lg2 = jnp.where(valid, logits_acc, jnp.float32(-1e9))
    m = jnp.max(lg2, axis=1, keepdims=True)
    e = jnp.exp(lg2 - m) * seg.astype(f32)
    attn = e / jnp.sum(e, axis=1, keepdims=True)
    out_ref[:] = (jnp.dot(attn, v_acc, preferred_element_type=f32)
                  + bout_ref[:])


def kernel(sensor_seq, mask, node_emb, edge_src, edge_dst, W_in, W_msg,
           W_self, w_att, W_out, b_out):
    x = sensor_seq.reshape(N_F, D_IN)
    es = edge_src.astype(jnp.int32)
    ed = edge_dst.astype(jnp.int32)
    es_row = es.reshape(C, 1, E_EDGES)
    es_col = es.reshape(C, E_EDGES, 1)
    ed_row = ed.reshape(C, 1, E_EDGES)
    ed_col = ed.reshape(C, E_EDGES, 1)
    watt = w_att.reshape(C, 1, D_HID)
    wout = W_out.reshape(C, D_HID, C)

    out = pl.pallas_call(
        _branch_kernel,
        out_shape=jax.ShapeDtypeStruct((B, C), jnp.float32),
        scratch_shapes=[
            pltpu.VMEM((N_NODES, 1), jnp.float32),
            pltpu.VMEM((N_NODES, 1), jnp.float32),
            pltpu.VMEM((N_NODES, D_HID), jnp.float32),
        ],
    )(x, node_emb, es_row, es_col, ed_row, ed_col, W_in, W_msg, W_self,
      watt, wout, mask.reshape(1, N_F), b_out.reshape(1, C))
    return out
